# baseline (speedup 1.0000x reference)
import jax
import jax.numpy as jnp
from jax.experimental import pallas as pl


def _matmul_body(x_ref, w_ref, b_ref, o_ref):
    acc = jax.lax.dot_general(
        x_ref[...], w_ref[...], (((1,), (1,)), ((), ())),
        preferred_element_type=jnp.float32,
    )
    o_ref[...] = acc + b_ref[...]


def kernel(input, sparse_weight, bias):
    M, K = input.shape
    N = sparse_weight.shape[0]
    bm, bn = 2048, 512
    xb = input.astype(jnp.bfloat16)
    wb = sparse_weight.astype(jnp.bfloat16)
    bias2 = bias.reshape(1, N)
    return pl.pallas_call(
        _matmul_body,
        grid=(M // bm, N // bn),
        in_specs=[
            pl.BlockSpec((bm, K), lambda m, n: (m, 0)),
            pl.BlockSpec((bn, K), lambda m, n: (n, 0)),
            pl.BlockSpec((1, bn), lambda m, n: (0, n)),
        ],
        out_specs=pl.BlockSpec((bm, bn), lambda m, n: (m, n)),
        out_shape=jax.ShapeDtypeStruct((M, N), jnp.float32),
    )(xb, wb, bias2)


# speedup vs baseline: 1.1862x; 1.1862x over previous
import jax
import jax.numpy as jnp
from jax.experimental import pallas as pl


def _matmul_body(x_ref, w_ref, b_ref, o_ref):
    xb = x_ref[...].astype(jnp.bfloat16)
    acc = jax.lax.dot_general(
        xb, w_ref[...], (((1,), (1,)), ((), ())),
        preferred_element_type=jnp.float32,
    )
    o_ref[...] = acc + b_ref[...]


def kernel(input, sparse_weight, bias):
    M, K = input.shape
    N = sparse_weight.shape[0]
    bm = 256
    wb = sparse_weight.astype(jnp.bfloat16)
    bias2 = bias.reshape(1, N)
    return pl.pallas_call(
        _matmul_body,
        grid=(M // bm,),
        in_specs=[
            pl.BlockSpec((bm, K), lambda m: (m, 0)),
            pl.BlockSpec((N, K), lambda m: (0, 0)),
            pl.BlockSpec((1, N), lambda m: (0, 0)),
        ],
        out_specs=pl.BlockSpec((bm, N), lambda m: (m, 0)),
        out_shape=jax.ShapeDtypeStruct((M, N), jnp.float32),
    )(input, wb, bias2)


# speedup vs baseline: 1.2365x; 1.0425x over previous
import jax
import jax.numpy as jnp
from jax.experimental import pallas as pl
from jax.experimental.pallas import tpu as pltpu


def kernel(input, sparse_weight, bias):
    M, K = input.shape
    N = sparse_weight.shape[0]
    bm = 256
    bc = 256
    nc = N // bc
    nm = M // bm
    half = N // 2
    p2 = nc // 2
    p3 = p2 + nm
    bias2 = bias.reshape(1, N)

    def _body(x_ref, w_ref, b_ref, o_ref, w_bf16):
        s = pl.program_id(0)

        @pl.when(s < nc)
        def _cast():
            w_bf16[pl.ds(s * bc, bc), :] = w_ref[...].astype(jnp.bfloat16)

        @pl.when(s >= p2)
        def _matmul():
            half_off = jnp.where(s < p3, 0, half)
            xb = x_ref[...].astype(jnp.bfloat16)
            wslice = w_bf16[pl.ds(half_off, half), :]
            acc = jax.lax.dot_general(
                xb, wslice, (((1,), (1,)), ((), ())),
                preferred_element_type=jnp.float32,
            )
            o_ref[...] = acc + b_ref[...]

    def x_idx(s):
        return (jnp.where(s < p3, jnp.clip(s - p2, 0, nm - 1), s - p3), 0)

    def w_idx(s):
        return (jnp.minimum(s, nc - 1), 0)

    def o_idx(s):
        return (jnp.where(s < p3, jnp.clip(s - p2, 0, nm - 1), s - p3),
                jnp.where(s < p3, 0, 1))

    def b_idx(s):
        return (0, jnp.where(s < p3, 0, 1))

    return pl.pallas_call(
        _body,
        grid=(p3 + nm,),
        in_specs=[
            pl.BlockSpec((bm, K), x_idx),
            pl.BlockSpec((bc, K), w_idx),
            pl.BlockSpec((1, half), b_idx),
        ],
        out_specs=pl.BlockSpec((bm, half), o_idx),
        out_shape=jax.ShapeDtypeStruct((M, N), jnp.float32),
        scratch_shapes=[pltpu.VMEM((N, K), jnp.bfloat16)],
    )(input, sparse_weight, bias2)


# speedup vs baseline: 1.2556x; 1.0154x over previous
import jax
import jax.numpy as jnp
from jax.experimental import pallas as pl
from jax.experimental.pallas import tpu as pltpu


def kernel(input, sparse_weight, bias):
    M, K = input.shape
    N = sparse_weight.shape[0]
    bm = 256
    bc = 256
    nc = N // bc
    nm = M // bm
    bias2 = bias.reshape(1, N)

    def _body(x_ref, w_ref, b_ref, o_ref, w_bf16):
        s = pl.program_id(0)

        @pl.when(s < nc)
        def _cast():
            w_bf16[pl.ds(s * bc, bc), :] = w_ref[...].astype(jnp.bfloat16)

        @pl.when(s >= nc)
        def _matmul():
            xb = x_ref[...].astype(jnp.bfloat16)
            acc = jax.lax.dot_general(
                xb, w_bf16[...], (((1,), (1,)), ((), ())),
                preferred_element_type=jnp.float32,
            )
            o_ref[...] = acc + b_ref[...]

    return pl.pallas_call(
        _body,
        grid=(nc + nm,),
        in_specs=[
            pl.BlockSpec((bm, K), lambda s: (jnp.clip(s - nc, 0, nm - 1), 0)),
            pl.BlockSpec((bc, K), lambda s: (jnp.minimum(s, nc - 1), 0)),
            pl.BlockSpec((1, N), lambda s: (0, 0)),
        ],
        out_specs=pl.BlockSpec((bm, N),
                               lambda s: (jnp.clip(s - nc, 0, nm - 1), 0)),
        out_shape=jax.ShapeDtypeStruct((M, N), jnp.float32),
        scratch_shapes=[pltpu.VMEM((N, K), jnp.bfloat16)],
    )(input, sparse_weight, bias2)
